# baseline (speedup 1.0000x reference)
import functools

import jax
import jax.numpy as jnp
from jax import lax
from jax.experimental import pallas as pl
from jax.experimental.pallas import tpu as pltpu
from jax.experimental.pallas import tpu_sc as plsc

N_NODES = 1000000
N_RELS = 1000
EMB = 64
BATCH = 16384
P_EPS = 1e-12

_NC = 2
_NS = 16
_NW = _NC * _NS
_BPW = BATCH // _NW
_GROUPS = _BPW // 16


def _tec_body(head_hbm, rel_hbm, tail_hbm, node_hbm, relemb_hbm, out_hbm,
              hidx_v, ridx_v, tidx_v, hrows_v, rrows_v, trows_v, out_v, sem):
    wid = lax.axis_index("s") * _NC + lax.axis_index("c")
    base = wid * _BPW

    pltpu.sync_copy(head_hbm.at[pl.ds(base, _BPW)], hidx_v)
    pltpu.sync_copy(rel_hbm.at[pl.ds(base, _BPW)], ridx_v)
    pltpu.sync_copy(tail_hbm.at[pl.ds(base, _BPW)], tidx_v)

    cp_h = pltpu.make_async_copy(node_hbm.at[hidx_v], hrows_v, sem)
    cp_r = pltpu.make_async_copy(relemb_hbm.at[ridx_v], rrows_v, sem)
    cp_t = pltpu.make_async_copy(node_hbm.at[tidx_v], trows_v, sem)
    cp_h.start()
    cp_r.start()
    cp_t.start()
    cp_h.wait()
    cp_r.wait()
    cp_t.wait()

    lane = lax.iota(jnp.int32, 16)
    perms = [lane ^ sh for sh in (8, 4, 2, 1)]

    def lanesum(x):
        for p in perms:
            x = x + x.at[p].get(mode="promise_in_bounds")
        return x

    def group(g, _):
        s_acc = jnp.zeros((16,), jnp.float32)
        c_acc = jnp.ones((16,), jnp.float32)
        for j in range(16):
            i = g * 16 + j
            h = [hrows_v[i, pl.ds(c * 16, 16)] for c in range(EMB // 16)]
            r = [rrows_v[i, pl.ds(c * 16, 16)] for c in range(EMB // 16)]
            t = [trows_v[i, pl.ds(c * 16, 16)] for c in range(EMB // 16)]
            na = jnp.abs(h[0]) + jnp.abs(h[1]) + jnp.abs(h[2]) + jnp.abs(h[3])
            nb = jnp.abs(t[0]) + jnp.abs(t[1]) + jnp.abs(t[2]) + jnp.abs(t[3])
            nh = jnp.maximum(lanesum(na), P_EPS)
            nt = jnp.maximum(lanesum(nb), P_EPS)
            c = nh * nt
            acc = jnp.zeros((16,), jnp.float32)
            for k in range(EMB // 16):
                acc = acc + jnp.abs(h[k] * nt + r[k] * c - t[k] * nh)
            s = lanesum(acc)
            s_acc = jnp.where(lane == j, s, s_acc)
            c_acc = jnp.where(lane == j, c, c_acc)
        out_v[pl.ds(g * 16, 16)] = s_acc / c_acc
        return _

    lax.fori_loop(0, _GROUPS, group, None)
    pltpu.sync_copy(out_v, out_hbm.at[pl.ds(base, _BPW)])


@jax.jit
def kernel(head_index, rel_type, tail_index, node_emb, rel_emb):
    mesh = plsc.VectorSubcoreMesh(core_axis_name="c", subcore_axis_name="s")
    f = pl.kernel(
        _tec_body,
        out_type=jax.ShapeDtypeStruct((BATCH,), jnp.float32),
        mesh=mesh,
        compiler_params=pltpu.CompilerParams(use_tc_tiling_on_sc=False),
        scratch_types=[
            pltpu.VMEM((_BPW,), jnp.int32),
            pltpu.VMEM((_BPW,), jnp.int32),
            pltpu.VMEM((_BPW,), jnp.int32),
            pltpu.VMEM((_BPW, EMB), jnp.float32),
            pltpu.VMEM((_BPW, EMB), jnp.float32),
            pltpu.VMEM((_BPW, EMB), jnp.float32),
            pltpu.VMEM((_BPW,), jnp.float32),
            pltpu.SemaphoreType.DMA,
        ],
    )
    return f(head_index, rel_type, tail_index, node_emb, rel_emb)


# speedup vs baseline: 2.3828x; 2.3828x over previous
import jax
import jax.numpy as jnp
from jax import lax
from jax.experimental import pallas as pl
from jax.experimental.pallas import tpu as pltpu
from jax.experimental.pallas import tpu_sc as plsc

N_NODES = 1000000
N_RELS = 1000
EMB = 64
BATCH = 16384
P_EPS = 1e-12

_NC = 2
_NS = 16
_NW = _NC * _NS
_BPW = BATCH // _NW
_CH = 32
_ROUNDS = _BPW // _CH


def _tec_body(head_hbm, rel_hbm, tail_hbm, node_hbm, relemb_hbm, dummy_hbm,
              out_hbm, sh_s, sr_s, st_s, vidx_v, stage_sh,
              hbuf0, hbuf1, rbuf0, rbuf1, tbuf0, tbuf1, out_v, sem0, sem1):
    wid = lax.axis_index("s") * _NC + lax.axis_index("c")
    base = wid * _BPW

    for k, (ihbm, smem) in enumerate(
            ((head_hbm, sh_s), (rel_hbm, sr_s), (tail_hbm, st_s))):
        pltpu.sync_copy(ihbm.at[pl.ds(base, _BPW)], vidx_v)
        pltpu.sync_copy(vidx_v, stage_sh.at[wid])
        pltpu.sync_copy(stage_sh.at[wid], smem)

    bufs = ((hbuf0, rbuf0, tbuf0), (hbuf1, rbuf1, tbuf1))
    sems = (sem0, sem1)

    def issue(r, b):
        def item(j, _):
            i = r * _CH + j
            ph = sh_s[i]
            pr = sr_s[i]
            pt = st_s[i]
            sem = sems[b]
            pltpu.make_async_copy(
                node_hbm.at[ph >> 3, ph & 7], bufs[b][0].at[j], sem).start()
            pltpu.make_async_copy(
                relemb_hbm.at[pr >> 3, pr & 7], bufs[b][1].at[j], sem).start()
            pltpu.make_async_copy(
                node_hbm.at[pt >> 3, pt & 7], bufs[b][2].at[j], sem).start()
            return _

        lax.fori_loop(0, _CH, item, None)

    def drain(b):
        def witem(j, _):
            for d in bufs[b]:
                pltpu.make_async_copy(dummy_hbm.at[0], d.at[j], sems[b]).wait()
            return _

        lax.fori_loop(0, _CH, witem, None)

    lane = lax.iota(jnp.int32, 16)
    perms = [lane ^ sh for sh in (8, 4, 2, 1)]

    def lanesum(x):
        for p in perms:
            x = x + x.at[p].get(mode="promise_in_bounds")
        return x

    def compute(r, b):
        hb, rb, tb = bufs[b]
        for g in range(_CH // 16):
            s_acc = jnp.zeros((16,), jnp.float32)
            c_acc = jnp.ones((16,), jnp.float32)
            for j in range(16):
                o = g * 16 + j
                h = [hb[o, pl.ds(c * 16, 16)] for c in range(4)]
                rr = [rb[o, pl.ds(c * 16, 16)] for c in range(4)]
                t = [tb[o, pl.ds(c * 16, 16)] for c in range(4)]
                na = (jnp.abs(h[0]) + jnp.abs(h[1])
                      + jnp.abs(h[2]) + jnp.abs(h[3]))
                nb = (jnp.abs(t[0]) + jnp.abs(t[1])
                      + jnp.abs(t[2]) + jnp.abs(t[3]))
                nh = jnp.maximum(lanesum(na), P_EPS)
                nt = jnp.maximum(lanesum(nb), P_EPS)
                c = nh * nt
                acc = jnp.zeros((16,), jnp.float32)
                for k in range(4):
                    acc = acc + jnp.abs(h[k] * nt + rr[k] * c - t[k] * nh)
                s = lanesum(acc)
                s_acc = jnp.where(lane == j, s, s_acc)
                c_acc = jnp.where(lane == j, c, c_acc)
            out_v[pl.ds(r * _CH + g * 16, 16)] = s_acc / c_acc

    issue(0, 0)

    def outer(rr, _):
        for b in range(2):
            r = rr * 2 + b

            @pl.when(r + 1 < _ROUNDS)
            def _issue():
                issue(r + 1, 1 - b)

            drain(b)
            compute(r, b)
        return _

    lax.fori_loop(0, _ROUNDS // 2, outer, None)
    pltpu.sync_copy(out_v, out_hbm.at[pl.ds(base, _BPW)])


@jax.jit
def kernel(head_index, rel_type, tail_index, node_emb, rel_emb):
    node3 = node_emb.reshape(N_NODES // 8, 8, EMB)
    rel3 = rel_emb.reshape(N_RELS // 8, 8, EMB)
    mesh = plsc.VectorSubcoreMesh(core_axis_name="c", subcore_axis_name="s")
    f = pl.kernel(
        _tec_body,
        out_type=jax.ShapeDtypeStruct((BATCH,), jnp.float32),
        mesh=mesh,
        compiler_params=pltpu.CompilerParams(
            use_tc_tiling_on_sc=True, needs_layout_passes=False),
        scratch_types=[
            pltpu.SMEM((_BPW,), jnp.int32),
            pltpu.SMEM((_BPW,), jnp.int32),
            pltpu.SMEM((_BPW,), jnp.int32),
            pltpu.VMEM((_BPW,), jnp.int32),
            pltpu.VMEM_SHARED((_NW, _BPW), jnp.int32),
            pltpu.VMEM((_CH, EMB), jnp.float32),
            pltpu.VMEM((_CH, EMB), jnp.float32),
            pltpu.VMEM((_CH, EMB), jnp.float32),
            pltpu.VMEM((_CH, EMB), jnp.float32),
            pltpu.VMEM((_CH, EMB), jnp.float32),
            pltpu.VMEM((_CH, EMB), jnp.float32),
            pltpu.VMEM((_BPW,), jnp.float32),
            pltpu.SemaphoreType.DMA,
            pltpu.SemaphoreType.DMA,
        ],
    )
    dummy = jnp.zeros((_CH, EMB), jnp.float32)
    return f(head_index, rel_type, tail_index, node3, rel3, dummy)
